# baseline (speedup 1.0000x reference)
import jax
import jax.numpy as jnp
from jax import lax
from jax.experimental import pallas as pl
from jax.experimental.pallas import tpu as pltpu, tpu_sc as plsc

_L = 16
_NC = 2
_NS = 16
_NW = _NC * _NS
_ROWS = 4096
_IN_COLS = 512
_OUT_COLS = 256
_ROWS_PER_W = _ROWS // _NW
_CH = 16
_NCHUNK = _ROWS_PER_W // _CH
_NBUF = 4
_GPR = _OUT_COLS // _L


def _sc_body(in_hbm, out_hbm, *refs):
    vins = refs[0:_NBUF]
    vouts = refs[_NBUF : _NBUF + 2]
    sins = refs[_NBUF + 2 : 2 * _NBUF + 2]
    souts = refs[2 * _NBUF + 2 : 2 * _NBUF + 4]

    wid = lax.axis_index("s") * _NC + lax.axis_index("c")
    base = wid * _ROWS_PER_W

    def start_in(c, b):
        return pltpu.async_copy(
            in_hbm.at[pl.ds(base + c * _CH, _CH)], vins[b], sins[b]
        )

    lane2 = 2 * lax.broadcasted_iota(jnp.int32, (_L,), 0)
    cols = [g * (2 * _L) + lane2 for g in range(_GPR)]

    in_flight = [start_in(c, c) for c in range(_NBUF - 1)] + [None]
    out_flight = [None, None]
    for c in range(_NCHUNK):
        b = c % _NBUF
        ob = c % 2
        in_flight[b].wait()
        nxt = c + _NBUF - 1
        if nxt < _NCHUNK:
            in_flight[nxt % _NBUF] = start_in(nxt, nxt % _NBUF)
        if out_flight[ob] is not None:
            out_flight[ob].wait()

        @plsc.parallel_loop(0, _CH, 1, unroll=4)
        def row_body(r, _b=b, _ob=ob):
            rows = jnp.broadcast_to(r, (_L,))
            for g in range(_GPR):
                vouts[_ob][r, pl.ds(g * _L, _L)] = plsc.load_gather(
                    vins[_b], [rows, cols[g]]
                )
        out_flight[ob] = pltpu.async_copy(
            vouts[ob], out_hbm.at[pl.ds(base + c * _CH, _CH)], souts[ob]
        )
    out_flight[0].wait()
    out_flight[1].wait()


_sc_deinterleave = pl.kernel(
    _sc_body,
    out_type=jax.ShapeDtypeStruct((_ROWS, _OUT_COLS), jnp.float32),
    mesh=plsc.VectorSubcoreMesh(
        core_axis_name="c", subcore_axis_name="s", num_cores=_NC, num_subcores=_NS
    ),
    scratch_types=(
        [pltpu.VMEM((_CH, _IN_COLS), jnp.float32) for _ in range(_NBUF)]
        + [pltpu.VMEM((_CH, _OUT_COLS), jnp.float32) for _ in range(2)]
        + [pltpu.SemaphoreType.DMA for _ in range(_NBUF + 2)]
    ),
    compiler_params=pltpu.CompilerParams(
        needs_layout_passes=False, skip_device_barrier=True
    ),
)


def kernel(firings):
    out = _sc_deinterleave(firings.reshape(_ROWS, _IN_COLS))
    return out.reshape(1, _ROWS, _OUT_COLS)


# speedup vs baseline: 1.0026x; 1.0026x over previous
import jax
import jax.numpy as jnp
from jax import lax
from jax.experimental import pallas as pl
from jax.experimental.pallas import tpu as pltpu, tpu_sc as plsc

_L = 16
_NC = 2
_NS = 16
_NW = _NC * _NS
_ROWS = 4096
_IN_COLS = 512
_OUT_COLS = 256
_ROWS_PER_W = _ROWS // _NW
_CH = 16
_NCHUNK = _ROWS_PER_W // _CH
_NBUF = 4
_GPR = _OUT_COLS // _L


def _sc_body(in_hbm, out_hbm, *refs):
    vins = refs[0:_NBUF]
    vouts = refs[_NBUF : _NBUF + 2]
    sins = refs[_NBUF + 2 : 2 * _NBUF + 2]
    souts = refs[2 * _NBUF + 2 : 2 * _NBUF + 4]

    wid = lax.axis_index("s") * _NC + lax.axis_index("c")
    base = wid * _ROWS_PER_W

    def start_in(c, b):
        return pltpu.async_copy(
            in_hbm.at[pl.ds(base + c * _CH, _CH)], vins[b], sins[b]
        )

    lane2 = 2 * lax.broadcasted_iota(jnp.int32, (_L,), 0)
    cols = [g * (2 * _L) + lane2 for g in range(_GPR)]

    in_flight = [start_in(c, c) for c in range(_NBUF - 1)] + [None]
    out_flight = [None, None]
    for c in range(_NCHUNK):
        b = c % _NBUF
        ob = c % 2
        in_flight[b].wait()
        nxt = c + _NBUF - 1
        if nxt < _NCHUNK:
            in_flight[nxt % _NBUF] = start_in(nxt, nxt % _NBUF)
        if out_flight[ob] is not None:
            out_flight[ob].wait()

        @plsc.parallel_loop(0, _CH, 1, unroll=4)
        def row_body(r, _b=b, _ob=ob):
            rows = jnp.broadcast_to(r, (_L,))
            for g in range(_GPR):
                vouts[_ob][r, pl.ds(g * _L, _L)] = plsc.load_gather(
                    vins[_b], [rows, cols[g]]
                )
        out_flight[ob] = pltpu.async_copy(
            vouts[ob], out_hbm.at[pl.ds(base + c * _CH, _CH)], souts[ob]
        )
    out_flight[0].wait()
    out_flight[1].wait()


_sc_deinterleave = pl.kernel(
    _sc_body,
    out_type=jax.ShapeDtypeStruct((_ROWS, _OUT_COLS), jnp.float32),
    mesh=plsc.VectorSubcoreMesh(
        core_axis_name="c", subcore_axis_name="s", num_cores=_NC, num_subcores=_NS
    ),
    scratch_types=(
        [pltpu.VMEM((_CH, _IN_COLS), jnp.float32) for _ in range(_NBUF)]
        + [pltpu.VMEM((_CH, _OUT_COLS), jnp.float32) for _ in range(2)]
        + [pltpu.SemaphoreType.DMA for _ in range(_NBUF + 2)]
    ),
    compiler_params=pltpu.CompilerParams(needs_layout_passes=False),
)


def kernel(firings):
    out = _sc_deinterleave(firings.reshape(_ROWS, _IN_COLS))
    return out.reshape(1, _ROWS, _OUT_COLS)
